# baseline (device time: 25829 ns/iter reference)
import jax
import jax.numpy as jnp
from jax import lax
from jax.experimental import pallas as pl
from jax.experimental.pallas import tpu as pltpu

N_DEV = 32
N_STEPS = 5


def _partners(my):
    z = my // 8
    p = my % 8
    y = p // 2
    x = (p + y) % 2

    def to_idx(xx, yy, zz):
        return 8 * zz + 2 * yy + (xx + yy) % 2

    return [
        to_idx(1 - x, y, z),
        to_idx(x, y ^ 1, z),
        to_idx(x, y, z ^ 1),
        to_idx(x, y ^ 2, z),
        to_idx(x, y, z ^ 2),
    ]


def kernel(x, W1, W2):
    m, k = x.shape
    _, h = W1.shape
    _, n = W2.shape
    nh = n // 2

    def body(x_ref, w1_ref, w2_ref, out_ref,
             acc_a, acc_b, recv_a, recv_b,
             send_sems_a, recv_sems_a, send_sems_b, recv_sems_b):
        my = lax.axis_index("i")
        partners = _partners(my)
        order_a = [0, 1, 2, 3, 4]
        order_b = [1, 0, 2, 3, 4]

        barrier_sem = pltpu.get_barrier_semaphore()
        for s in range(N_STEPS):
            pl.semaphore_signal(
                barrier_sem, inc=1,
                device_id=(partners[s],),
                device_id_type=pl.DeviceIdType.MESH,
            )

        xb = x_ref[...].astype(jnp.bfloat16)
        w1b = w1_ref[...].astype(jnp.bfloat16)
        w2b = w2_ref[...].astype(jnp.bfloat16)
        hid = jnp.dot(xb, w1b, preferred_element_type=jnp.float32)
        hid = jnp.maximum(hid, 0.0).astype(jnp.bfloat16)
        acc_a[...] = jnp.dot(
            hid, w2b[:, :nh], preferred_element_type=jnp.float32
        ).astype(jnp.bfloat16)

        pl.semaphore_wait(barrier_sem, N_STEPS)

        def make(chain, t):
            acc, recv, ssems, rsems, order = (
                (acc_a, recv_a, send_sems_a, recv_sems_a, order_a)
                if chain == 0
                else (acc_b, recv_b, send_sems_b, recv_sems_b, order_b)
            )
            return pltpu.make_async_remote_copy(
                src_ref=acc,
                dst_ref=recv.at[t],
                send_sem=ssems.at[t],
                recv_sem=rsems.at[t],
                device_id=(partners[order[t]],),
                device_id_type=pl.DeviceIdType.MESH,
            )

        rdma_a = make(0, 0)
        rdma_a.start()
        acc_b[...] = jnp.dot(
            hid, w2b[:, nh:], preferred_element_type=jnp.float32
        ).astype(jnp.bfloat16)
        rdma_b = make(1, 0)
        rdma_b.start()

        for t in range(N_STEPS):
            rdma_a.wait()
            acc_a[...] = acc_a[...] + recv_a[t]
            if t + 1 < N_STEPS:
                rdma_a = make(0, t + 1)
                rdma_a.start()
            rdma_b.wait()
            acc_b[...] = acc_b[...] + recv_b[t]
            if t + 1 < N_STEPS:
                rdma_b = make(1, t + 1)
                rdma_b.start()

        out_ref[:, :nh] = acc_a[...].astype(jnp.float32)
        out_ref[:, nh:] = acc_b[...].astype(jnp.float32)

    return pl.pallas_call(
        body,
        out_shape=jax.ShapeDtypeStruct((m, n), jnp.float32),
        in_specs=[pl.BlockSpec(memory_space=pltpu.VMEM)] * 3,
        out_specs=pl.BlockSpec(memory_space=pltpu.VMEM),
        scratch_shapes=[
            pltpu.VMEM((m, nh), jnp.bfloat16),
            pltpu.VMEM((m, nh), jnp.bfloat16),
            pltpu.VMEM((N_STEPS, m, nh), jnp.bfloat16),
            pltpu.VMEM((N_STEPS, m, nh), jnp.bfloat16),
            pltpu.SemaphoreType.DMA((N_STEPS,)),
            pltpu.SemaphoreType.DMA((N_STEPS,)),
            pltpu.SemaphoreType.DMA((N_STEPS,)),
            pltpu.SemaphoreType.DMA((N_STEPS,)),
        ],
        compiler_params=pltpu.CompilerParams(collective_id=0),
    )(x, W1, W2)


# device time: 21791 ns/iter; 1.1853x vs baseline; 1.1853x over previous
import jax
import jax.numpy as jnp
from jax import lax
from jax.experimental import pallas as pl
from jax.experimental.pallas import tpu as pltpu

N_DEV = 32
N_STEPS = 5
N_CHAINS = 8


def _partners(my):
    z = my // 8
    p = my % 8
    y = p // 2
    x = (p + y) % 2

    def to_idx(xx, yy, zz):
        return 8 * zz + 2 * yy + (xx + yy) % 2

    return [
        to_idx(1 - x, y, z),
        to_idx(x, y ^ 1, z),
        to_idx(x, y, z ^ 1),
        to_idx(x, y ^ 2, z),
        to_idx(x, y, z ^ 2),
    ]


def kernel(x, W1, W2):
    m, k = x.shape
    _, h = W1.shape
    _, n = W2.shape
    mc = m // N_CHAINS

    def body(x_ref, w1_ref, w2_ref, out_ref, acc_ref, recv_ref,
             send_sems, recv_sems):
        my = lax.axis_index("i")
        partners = _partners(my)

        barrier_sem = pltpu.get_barrier_semaphore()
        for s in range(N_STEPS):
            pl.semaphore_signal(
                barrier_sem, inc=1,
                device_id=(partners[s],),
                device_id_type=pl.DeviceIdType.MESH,
            )

        xb = x_ref[...].astype(jnp.bfloat16)
        w1b = w1_ref[...].astype(jnp.bfloat16)
        w2b = w2_ref[...].astype(jnp.bfloat16)

        def make(c, t):
            return pltpu.make_async_remote_copy(
                src_ref=acc_ref.at[c],
                dst_ref=recv_ref.at[c, t],
                send_sem=send_sems.at[c, t],
                recv_sem=recv_sems.at[c, t],
                device_id=(partners[(t + c) % N_STEPS],),
                device_id_type=pl.DeviceIdType.MESH,
            )

        rdmas = [None] * N_CHAINS
        for c in range(N_CHAINS):
            hid_c = jnp.dot(
                xb[c * mc:(c + 1) * mc], w1b,
                preferred_element_type=jnp.float32,
            )
            hid_c = jnp.maximum(hid_c, 0.0).astype(jnp.bfloat16)
            acc_ref[c] = jnp.dot(
                hid_c, w2b, preferred_element_type=jnp.float32
            ).astype(jnp.bfloat16)
            if c == 0:
                pl.semaphore_wait(barrier_sem, N_STEPS)
            rdmas[c] = make(c, 0)
            rdmas[c].start()

        for t in range(N_STEPS):
            for c in range(N_CHAINS):
                rdmas[c].wait()
                if t + 1 < N_STEPS:
                    acc_ref[c] = acc_ref[c] + recv_ref[c, t]
                    rdmas[c] = make(c, t + 1)
                    rdmas[c].start()
                else:
                    out_ref[c * mc:(c + 1) * mc, :] = (
                        acc_ref[c] + recv_ref[c, t]
                    )

    return pl.pallas_call(
        body,
        out_shape=jax.ShapeDtypeStruct((m, n), jnp.bfloat16),
        in_specs=[pl.BlockSpec(memory_space=pltpu.VMEM)] * 3,
        out_specs=pl.BlockSpec(memory_space=pltpu.VMEM),
        scratch_shapes=[
            pltpu.VMEM((N_CHAINS, mc, n), jnp.bfloat16),
            pltpu.VMEM((N_CHAINS, N_STEPS, mc, n), jnp.bfloat16),
            pltpu.SemaphoreType.DMA((N_CHAINS, N_STEPS)),
            pltpu.SemaphoreType.DMA((N_CHAINS, N_STEPS)),
        ],
        compiler_params=pltpu.CompilerParams(collective_id=0),
    )(x, W1, W2)
